# baseline (speedup 1.0000x reference)
import jax
import jax.numpy as jnp
from jax import lax
from jax.experimental import pallas as pl
from jax.experimental.pallas import tpu as pltpu
from jax.experimental.pallas import tpu_sc as plsc

B = 16
Z = 9
CH = 256
X = 32
Y = 32
CHANNELS = 86
L = 16

_OFF_COL = 0
_OFF_ROW = X * CHANNELS
_OFF_HEI = 2 * X * CHANNELS
_HEI_PAD = 776
_W_SIZE = _OFF_HEI + _HEI_PAD

_PAIRS = 9
_GROUP = 3


def _sc_body(col_hbm, row_hbm, hei_hbm, out_hbm, w_v, buf, sem):
    nc = 2
    wid = lax.axis_index("s") * nc + lax.axis_index("c")

    pltpu.sync_copy(col_hbm, w_v.at[pl.ds(_OFF_COL, _OFF_ROW)])
    pltpu.sync_copy(row_hbm, w_v.at[pl.ds(_OFF_ROW, _OFF_ROW)])
    pltpu.sync_copy(hei_hbm, w_v.at[pl.ds(_OFF_HEI, _HEI_PAD)])

    lane = lax.iota(jnp.int32, L)
    row0 = wid * _PAIRS * X

    waves = {}
    for t in range(_PAIRS):
        p = wid * _PAIRS + t
        z = p >> 5
        i = p & 31
        row_base = _OFF_ROW - CHANNELS + i * CHANNELS
        hei_base = _OFF_HEI - 2 * CHANNELS + z * CHANNELS

        def _j_body(j, _, t=t, row_base=row_base, hei_base=hei_base):
            col_base = j * CHANNELS
            row = t * X + j
            for lg in range(CH // L):
                c0 = lg * L
                cvec = lane + c0
                if c0 + L <= CHANNELS:
                    v = w_v[pl.ds(col_base + c0, L)]
                elif c0 >= CHANNELS and c0 + L <= 2 * CHANNELS:
                    v = w_v[pl.ds(row_base + c0, L)]
                elif c0 >= 2 * CHANNELS:
                    v = w_v[pl.ds(hei_base + c0, L)]
                elif c0 < CHANNELS:
                    va = w_v[pl.ds(col_base + c0, L)]
                    vb = w_v[pl.ds(row_base + c0, L)]
                    v = jnp.where(cvec < CHANNELS, va, vb)
                else:
                    va = w_v[pl.ds(row_base + c0, L)]
                    vb = w_v[pl.ds(hei_base + c0, L)]
                    v = jnp.where(cvec < 2 * CHANNELS, va, vb)
                buf[row, pl.ds(c0, L)] = v
            return 0

        lax.fori_loop(0, X, _j_body, 0)

        if t % _GROUP == _GROUP - 1:
            g = t // _GROUP
            rows = _GROUP * X
            waves[g] = [
                pltpu.async_copy(
                    buf.at[pl.ds(g * rows, rows), :],
                    out_hbm.at[bb, pl.ds(row0 + g * rows, rows), :],
                    sem,
                )
                for bb in range(B)
            ]
            if g > 0:
                for cp in waves[g - 1]:
                    cp.wait()
    for cp in waves[_PAIRS // _GROUP - 1]:
        cp.wait()


def kernel(tensor, row_w, col_w, hei_w):
    del tensor
    mesh = plsc.VectorSubcoreMesh(core_axis_name="c", subcore_axis_name="s")
    run = pl.kernel(
        _sc_body,
        out_type=jax.ShapeDtypeStruct((B, Z * X * Y, CH), jnp.float32),
        mesh=mesh,
        scratch_types=[
            pltpu.VMEM((_W_SIZE,), jnp.float32),
            pltpu.VMEM((_PAIRS * X, CH), jnp.float32),
            pltpu.SemaphoreType.DMA,
        ],
    )
    out = run(
        col_w.reshape(-1),
        row_w.reshape(-1),
        jnp.pad(hei_w.reshape(-1), (0, _HEI_PAD - Z * CHANNELS)),
    )
    return out.reshape(B, Z, X, Y, CH).transpose(0, 1, 4, 2, 3)


# speedup vs baseline: 1.6358x; 1.6358x over previous
import jax
import jax.numpy as jnp
from jax.experimental import pallas as pl
from jax.experimental.pallas import tpu as pltpu

B = 16
Z = 9
CH = 256
X = 32
Y = 32
CHANNELS = 86
C_REST = CH - 2 * CHANNELS


def _pos_body(col_ref, row_ref, hei_ref, out_ref, scratch, cr_s, sem):
    z = pl.program_id(0)
    ci = jax.lax.broadcasted_iota(jnp.int32, (X, Y, CH), 2)

    @pl.when(z == 0)
    def _():
        zeros = jnp.zeros((X, CHANNELS), jnp.float32)
        cw = jnp.concatenate(
            [col_ref[...], zeros, zeros[:, :C_REST]], axis=1)
        rw = jnp.concatenate(
            [zeros, row_ref[...], zeros[:, :C_REST]], axis=1)
        a = jnp.broadcast_to(cw[None, :, :], (X, Y, CH))
        b = jnp.broadcast_to(rw[:, None, :], (X, Y, CH))
        cr_s[...] = jnp.where(ci < CHANNELS, a, b)

    hz = hei_ref[pl.ds(z, 1), :]
    hw = jnp.concatenate(
        [jnp.zeros((1, 2 * CHANNELS), jnp.float32), hz[:, :C_REST]], axis=1)
    c = jnp.broadcast_to(hw[0][None, None, :], (X, Y, CH))
    scratch[pl.ds(z, 1)] = jnp.where(ci < 2 * CHANNELS, cr_s[...], c)[None]

    def _copies(zz):
        return [
            pltpu.make_async_copy(
                scratch.at[pl.ds(zz, 1)],
                out_ref.at[bb, pl.ds(zz, 1)],
                sem,
            )
            for bb in range(B)
        ]

    for cp in _copies(z):
        cp.start()

    @pl.when(z > 0)
    def _():
        for cp in _copies(z - 1):
            cp.wait()

    @pl.when(z == Z - 1)
    def _():
        for cp in _copies(z):
            cp.wait()


def kernel(tensor, row_w, col_w, hei_w):
    del tensor
    out = pl.pallas_call(
        _pos_body,
        grid=(Z,),
        in_specs=[
            pl.BlockSpec((Y, CHANNELS), lambda z: (0, 0)),
            pl.BlockSpec((X, CHANNELS), lambda z: (0, 0)),
            pl.BlockSpec((Z, CHANNELS), lambda z: (0, 0)),
        ],
        out_specs=pl.BlockSpec(memory_space=pl.ANY),
        out_shape=jax.ShapeDtypeStruct((B, Z, X, Y, CH), jnp.float32),
        scratch_shapes=[
            pltpu.VMEM((Z, X, Y, CH), jnp.float32),
            pltpu.VMEM((X, Y, CH), jnp.float32),
            pltpu.SemaphoreType.DMA,
        ],
        compiler_params=pltpu.CompilerParams(
            dimension_semantics=("arbitrary",),
        ),
    )(col_w, row_w, hei_w)
    return jnp.transpose(out, (0, 1, 4, 2, 3))


# speedup vs baseline: 1.6360x; 1.0001x over previous
import jax
import jax.numpy as jnp
from jax.experimental import pallas as pl
from jax.experimental.pallas import tpu as pltpu

B = 16
Z = 9
CH = 256
X = 32
Y = 32
CHANNELS = 86
C_REST = CH - 2 * CHANNELS


def _pos_body(col_ref, row_ref, hei_ref, out_ref, scratch, cr_s, sem):
    z = pl.program_id(0)
    ci = jax.lax.broadcasted_iota(jnp.int32, (X, Y, CH), 2)

    @pl.when(z == 0)
    def _():
        zeros = jnp.zeros((X, CHANNELS), jnp.float32)
        cw = jnp.concatenate(
            [col_ref[...], zeros, zeros[:, :C_REST]], axis=1)
        rw = jnp.concatenate(
            [zeros, row_ref[...], zeros[:, :C_REST]], axis=1)
        a = jnp.broadcast_to(cw[None, :, :], (X, Y, CH))
        b = jnp.broadcast_to(rw[:, None, :], (X, Y, CH))
        cr_s[...] = jnp.where(ci < CHANNELS, a, b)

    hz = hei_ref[pl.ds(z, 1), :]
    hw = jnp.concatenate(
        [jnp.zeros((1, 2 * CHANNELS), jnp.float32), hz[:, :C_REST]], axis=1)
    c = jnp.broadcast_to(hw[0][None, None, :], (X, Y, CH))
    scratch[pl.ds(z, 1)] = jnp.where(ci < 2 * CHANNELS, cr_s[...], c)[None]

    def _copies(zz):
        return [
            pltpu.make_async_copy(
                scratch.at[pl.ds(zz, 1), pl.ds(hh * (X // 2), X // 2)],
                out_ref.at[bb, pl.ds(zz, 1), pl.ds(hh * (X // 2), X // 2)],
                sem,
            )
            for bb in range(B)
            for hh in range(2)
        ]

    for cp in _copies(z):
        cp.start()

    @pl.when(z > 0)
    def _():
        for cp in _copies(z - 1):
            cp.wait()

    @pl.when(z == Z - 1)
    def _():
        for cp in _copies(z):
            cp.wait()


def kernel(tensor, row_w, col_w, hei_w):
    del tensor
    out = pl.pallas_call(
        _pos_body,
        grid=(Z,),
        in_specs=[
            pl.BlockSpec((Y, CHANNELS), lambda z: (0, 0)),
            pl.BlockSpec((X, CHANNELS), lambda z: (0, 0)),
            pl.BlockSpec((Z, CHANNELS), lambda z: (0, 0)),
        ],
        out_specs=pl.BlockSpec(memory_space=pl.ANY),
        out_shape=jax.ShapeDtypeStruct((B, Z, X, Y, CH), jnp.float32),
        scratch_shapes=[
            pltpu.VMEM((Z, X, Y, CH), jnp.float32),
            pltpu.VMEM((X, Y, CH), jnp.float32),
            pltpu.SemaphoreType.DMA,
        ],
        compiler_params=pltpu.CompilerParams(
            dimension_semantics=("arbitrary",),
        ),
    )(col_w, row_w, hei_w)
    return jnp.transpose(out, (0, 1, 4, 2, 3))
